# baseline (speedup 1.0000x reference)
import jax
import jax.numpy as jnp
from jax import lax
from jax.experimental import pallas as pl
from jax.experimental.pallas import tpu as pltpu

_EPS = 1e-5
_N = 16

_TAPMAP = {0: (3, 1), 1: (2, 0)}


def _bn_relu(ys, g, b, m_real):
    c = ys[0].shape[-1]
    s = jnp.zeros((1, c), jnp.float32)
    sq = jnp.zeros((1, c), jnp.float32)
    for y in ys:
        s = s + jnp.sum(y, axis=0, keepdims=True)
        sq = sq + jnp.sum(y * y, axis=0, keepdims=True)
    inv_m = 1.0 / m_real
    mean = s * inv_m
    var = sq * inv_m - mean * mean
    scale = g * lax.rsqrt(var + _EPS)
    shift = b - mean * scale
    return [jnp.maximum(y * scale + shift, 0.0) for y in ys]


def _pad_hw(x):
    h, w, n, c = x.shape
    zr = jnp.zeros((1, w, n, c), x.dtype)
    x = jnp.concatenate([zr, x, zr], axis=0)
    zc = jnp.zeros((h + 2, 1, n, c), x.dtype)
    return jnp.concatenate([zc, x, zc], axis=1)


def _parity_patches(xp, ph, pw, h, w):
    c = xp.shape[-1]
    taps = [xp[ph + dh:ph + dh + h, pw + dw:pw + dw + w].reshape(h * w * _N, c)
            for dh in (0, 1) for dw in (0, 1)]
    return jnp.concatenate(taps, axis=-1)


def _gen_kernel(x_ref, w1_hbm, w2_hbm, w3_hbm, w4_hbm, w5_hbm, gb_ref, o_ref,
                w1_v, w2_v, w3_v, w4_v, w5_v, sem):
    cps = []
    for i, (hbm, vm) in enumerate(((w1_hbm, w1_v), (w2_hbm, w2_v),
                                   (w3_hbm, w3_v), (w4_hbm, w4_v),
                                   (w5_hbm, w5_v))):
        cp = pltpu.make_async_copy(hbm, vm, sem.at[i])
        cp.start()
        cps.append(cp)

    x0 = x_ref[...].astype(jnp.bfloat16)

    cps[0].wait()
    ys = [jnp.dot(x0, w1_v[i], preferred_element_type=jnp.float32)
          for i in range(16)]
    ys = _bn_relu(ys, gb_ref[0:1, :512], gb_ref[1:2, :512], float(_N * 16))
    x = jnp.stack(ys, axis=0).reshape(4, 4, _N, 512).astype(jnp.bfloat16)

    for li, (w_v, (h, w, co)) in enumerate((
            (w2_v, (4, 4, 256)),
            (w3_v, (8, 8, 128)),
            (w4_v, (16, 16, 64)))):
        cps[li + 1].wait()
        xp = _pad_hw(x)
        yps = []
        for ph in (0, 1):
            for pw in (0, 1):
                a = _parity_patches(xp, ph, pw, h, w)
                yps.append(jnp.dot(a, w_v[2 * ph + pw],
                                   preferred_element_type=jnp.float32))
        g = gb_ref[2 * li + 2:2 * li + 3, :co]
        b = gb_ref[2 * li + 3:2 * li + 4, :co]
        yps = _bn_relu(yps, g, b, float(4 * h * w * _N))
        t = [y.reshape(h, w, _N, co) for y in yps]
        top = jnp.stack([t[0], t[1]], axis=2).reshape(h, 2 * w, _N, co)
        bot = jnp.stack([t[2], t[3]], axis=2).reshape(h, 2 * w, _N, co)
        x = (jnp.stack([top, bot], axis=1)
             .reshape(2 * h, 2 * w, _N, co).astype(jnp.bfloat16))

    cps[4].wait()
    xp = _pad_hw(x)
    for ph in (0, 1):
        for pw in (0, 1):
            a = _parity_patches(xp, ph, pw, 32, 32)
            y = jnp.dot(a, w5_v[2 * ph + pw],
                        preferred_element_type=jnp.float32)
            o_ref[2 * ph + pw] = jnp.tanh(y)


def _prep_s2_weights(w, cpad=None):
    bs = []
    for ph in (0, 1):
        for pw in (0, 1):
            bs.append(jnp.concatenate(
                [w[:, :, _TAPMAP[ph][dh], _TAPMAP[pw][dw]]
                 for dh in (0, 1) for dw in (0, 1)], axis=0))
    b = jnp.stack(bs, axis=0)
    if cpad is not None and cpad != b.shape[-1]:
        b = jnp.pad(b, ((0, 0), (0, 0), (0, cpad - b.shape[-1])))
    return b.astype(jnp.bfloat16)


def kernel(x_nchw, w1, w2, w3, w4, w5, g1, b1, g2, b2, g3, b3, g4, b4):
    x0 = x_nchw.reshape(_N, 256)
    w1p = jnp.stack([w1[:, :, i // 4, i % 4] for i in range(16)],
                    axis=0).astype(jnp.bfloat16)
    w2p = _prep_s2_weights(w2)
    w3p = _prep_s2_weights(w3)
    w4p = _prep_s2_weights(w4)
    w5p = _prep_s2_weights(w5, cpad=8)
    gb = jnp.stack([jnp.pad(a.astype(jnp.float32), (0, 512 - a.shape[0]))
                    for a in (g1, b1, g2, b2, g3, b3, g4, b4)], axis=0)

    any_spec = pl.BlockSpec(memory_space=pl.ANY)
    y = pl.pallas_call(
        _gen_kernel,
        out_shape=jax.ShapeDtypeStruct((4, 32 * 32 * _N, 8), jnp.float32),
        in_specs=[pl.BlockSpec((_N, 256), lambda: (0, 0)),
                  any_spec, any_spec, any_spec, any_spec, any_spec,
                  pl.BlockSpec((8, 512), lambda: (0, 0))],
        out_specs=pl.BlockSpec((4, 32 * 32 * _N, 8), lambda: (0, 0, 0)),
        scratch_shapes=[
            pltpu.VMEM((16, 256, 512), jnp.bfloat16),
            pltpu.VMEM((4, 2048, 256), jnp.bfloat16),
            pltpu.VMEM((4, 1024, 128), jnp.bfloat16),
            pltpu.VMEM((4, 512, 64), jnp.bfloat16),
            pltpu.VMEM((4, 256, 8), jnp.bfloat16),
            pltpu.SemaphoreType.DMA((5,)),
        ],
        compiler_params=pltpu.CompilerParams(
            vmem_limit_bytes=56 * 1024 * 1024),
    )(x0, w1p, w2p, w3p, w4p, w5p, gb)

    img = y[:, :, 0].reshape(2, 2, 32, 32, _N)
    img = jnp.transpose(img, (4, 2, 0, 3, 1)).reshape(_N, 64, 64)
    return img[:, None, :, :]


# speedup vs baseline: 1.0866x; 1.0866x over previous
import jax
import jax.numpy as jnp
from jax import lax
from jax.experimental import pallas as pl
from jax.experimental.pallas import tpu as pltpu

_EPS = 1e-5
_N = 16

_TAPMAP = {0: (3, 1), 1: (2, 0)}


def _bn_relu(ys, g, b, m_real):
    c = ys[0].shape[-1]
    s = jnp.zeros((1, c), jnp.float32)
    sq = jnp.zeros((1, c), jnp.float32)
    for y in ys:
        s = s + jnp.sum(y, axis=0, keepdims=True)
        sq = sq + jnp.sum(y * y, axis=0, keepdims=True)
    inv_m = 1.0 / m_real
    mean = s * inv_m
    var = sq * inv_m - mean * mean
    scale = g * lax.rsqrt(var + _EPS)
    shift = b - mean * scale
    return [jnp.maximum(y * scale + shift, 0.0) for y in ys]


def _pad_hw(x):
    h, w, n, c = x.shape
    zr = jnp.zeros((1, w, n, c), x.dtype)
    x = jnp.concatenate([zr, x, zr], axis=0)
    zc = jnp.zeros((h + 2, 1, n, c), x.dtype)
    return jnp.concatenate([zc, x, zc], axis=1)


def _parity_patches(xp, ph, pw, h, w):
    c = xp.shape[-1]
    taps = [xp[ph + dh:ph + dh + h, pw + dw:pw + dw + w].reshape(h * w * _N, c)
            for dh in (0, 1) for dw in (0, 1)]
    return jnp.concatenate(taps, axis=-1)


def _gen_kernel(x_ref, w1_ref, w2_ref, w3_ref, w4_ref, w5_ref, gb_ref, o_ref):
    x0 = x_ref[...].astype(jnp.bfloat16)

    ys = [jnp.dot(x0, w1_ref[i], preferred_element_type=jnp.float32)
          for i in range(16)]
    ys = _bn_relu(ys, gb_ref[0:1, :512], gb_ref[1:2, :512], float(_N * 16))
    x = jnp.stack(ys, axis=0).reshape(4, 4, _N, 512).astype(jnp.bfloat16)

    for li, (w_ref, (h, w, co)) in enumerate((
            (w2_ref, (4, 4, 256)),
            (w3_ref, (8, 8, 128)),
            (w4_ref, (16, 16, 64)))):
        xp = _pad_hw(x)
        yps = []
        for ph in (0, 1):
            for pw in (0, 1):
                a = _parity_patches(xp, ph, pw, h, w)
                yps.append(jnp.dot(a, w_ref[2 * ph + pw],
                                   preferred_element_type=jnp.float32))
        g = gb_ref[2 * li + 2:2 * li + 3, :co]
        b = gb_ref[2 * li + 3:2 * li + 4, :co]
        yps = _bn_relu(yps, g, b, float(4 * h * w * _N))
        t = [y.reshape(h, w, _N, co) for y in yps]
        top = jnp.stack([t[0], t[1]], axis=2).reshape(h, 2 * w, _N, co)
        bot = jnp.stack([t[2], t[3]], axis=2).reshape(h, 2 * w, _N, co)
        x = (jnp.stack([top, bot], axis=1)
             .reshape(2 * h, 2 * w, _N, co).astype(jnp.bfloat16))

    xp = _pad_hw(x)
    for ph in (0, 1):
        for pw in (0, 1):
            a = _parity_patches(xp, ph, pw, 32, 32)
            y = jnp.dot(a, w5_ref[2 * ph + pw],
                        preferred_element_type=jnp.float32)
            o_ref[2 * ph + pw] = jnp.tanh(y).astype(jnp.bfloat16)


def _prep_s2_weights(w, cpad=None):
    bs = []
    for ph in (0, 1):
        for pw in (0, 1):
            bs.append(jnp.concatenate(
                [w[:, :, _TAPMAP[ph][dh], _TAPMAP[pw][dw]]
                 for dh in (0, 1) for dw in (0, 1)], axis=0))
    b = jnp.stack(bs, axis=0)
    if cpad is not None and cpad != b.shape[-1]:
        b = jnp.pad(b, ((0, 0), (0, 0), (0, cpad - b.shape[-1])))
    return b.astype(jnp.bfloat16)


def kernel(x_nchw, w1, w2, w3, w4, w5, g1, b1, g2, b2, g3, b3, g4, b4):
    x0 = x_nchw.reshape(_N, 256)
    w1p = (jnp.transpose(w1, (2, 3, 0, 1)).astype(jnp.bfloat16)
           .reshape(16, 256, 512))
    w2p = _prep_s2_weights(w2)
    w3p = _prep_s2_weights(w3)
    w4p = _prep_s2_weights(w4)
    w5p = _prep_s2_weights(w5, cpad=8)
    gb = jnp.stack([jnp.pad(a.astype(jnp.float32), (0, 512 - a.shape[0]))
                    for a in (g1, b1, g2, b2, g3, b3, g4, b4)], axis=0)

    y = pl.pallas_call(
        _gen_kernel,
        out_shape=jax.ShapeDtypeStruct((4, 32 * 32 * _N, 8), jnp.bfloat16),
        compiler_params=pltpu.CompilerParams(
            vmem_limit_bytes=56 * 1024 * 1024),
    )(x0, w1p, w2p, w3p, w4p, w5p, gb)

    img = jnp.sum(y.astype(jnp.float32), axis=-1).reshape(2, 2, 32, 32, _N)
    img = jnp.transpose(img, (4, 2, 0, 3, 1)).reshape(_N, 64, 64)
    return img[:, None, :, :]


# speedup vs baseline: 1.4544x; 1.3384x over previous
import jax
import jax.numpy as jnp
from jax import lax
from jax.experimental import pallas as pl
from jax.experimental.pallas import tpu as pltpu

_EPS = 1e-5
_N = 16

_TAPMAP = {0: (3, 1), 1: (2, 0)}


def _bn_relu(ys, g, b, m_real):
    c = ys[0].shape[-1]
    s = jnp.zeros((1, c), jnp.float32)
    sq = jnp.zeros((1, c), jnp.float32)
    for y in ys:
        s = s + jnp.sum(y, axis=0, keepdims=True)
        sq = sq + jnp.sum(y * y, axis=0, keepdims=True)
    inv_m = 1.0 / m_real
    mean = s * inv_m
    var = sq * inv_m - mean * mean
    scale = g * lax.rsqrt(var + _EPS)
    shift = b - mean * scale
    return [jnp.maximum(y * scale + shift, 0.0) for y in ys]


def _pad_hw(x):
    h, w, n, c = x.shape
    zr = jnp.zeros((1, w, n, c), x.dtype)
    x = jnp.concatenate([zr, x, zr], axis=0)
    zc = jnp.zeros((h + 2, 1, n, c), x.dtype)
    return jnp.concatenate([zc, x, zc], axis=1)


def _parity_patches(xp, ph, pw, h, w):
    c = xp.shape[-1]
    taps = [xp[ph + dh:ph + dh + h, pw + dw:pw + dw + w].reshape(h * w * _N, c)
            for dh in (0, 1) for dw in (0, 1)]
    return jnp.concatenate(taps, axis=-1)


def _gen_kernel(x_ref, w1_ref, w2_ref, w3_ref, w4_ref, w5_ref, gb_ref, o_ref):
    x0 = x_ref[...].astype(jnp.bfloat16)

    ys = [jnp.dot(x0, w1_ref[i], preferred_element_type=jnp.float32)
          for i in range(16)]
    ys = _bn_relu(ys, gb_ref[0:1, :512], gb_ref[1:2, :512], float(_N * 16))
    x = jnp.stack(ys, axis=0).reshape(4, 4, _N, 512).astype(jnp.bfloat16)

    for li, (w_ref, (h, w, co)) in enumerate((
            (w2_ref, (4, 4, 256)),
            (w3_ref, (8, 8, 128)),
            (w4_ref, (16, 16, 64)))):
        xp = _pad_hw(x)
        yps = []
        for ph in (0, 1):
            for pw in (0, 1):
                a = _parity_patches(xp, ph, pw, h, w)
                yps.append(jnp.dot(a, w_ref[2 * ph + pw],
                                   preferred_element_type=jnp.float32))
        g = gb_ref[2 * li + 2:2 * li + 3, :co]
        b = gb_ref[2 * li + 3:2 * li + 4, :co]
        yps = _bn_relu(yps, g, b, float(4 * h * w * _N))
        t = [y.reshape(h, w, _N, co) for y in yps]
        top = jnp.stack([t[0], t[1]], axis=2).reshape(h, 2 * w, _N, co)
        bot = jnp.stack([t[2], t[3]], axis=2).reshape(h, 2 * w, _N, co)
        x = (jnp.stack([top, bot], axis=1)
             .reshape(2 * h, 2 * w, _N, co).astype(jnp.bfloat16))

    xp = _pad_hw(x)
    for ph in (0, 1):
        for pw in (0, 1):
            a = _parity_patches(xp, ph, pw, 32, 32)
            y = jnp.dot(a, w5_ref[2 * ph + pw],
                        preferred_element_type=jnp.float32)
            o_ref[2 * ph + pw] = jnp.tanh(jnp.transpose(y))


def _prep_s2_weights(w, cpad=None):
    bs = []
    for ph in (0, 1):
        for pw in (0, 1):
            bs.append(jnp.concatenate(
                [w[:, :, _TAPMAP[ph][dh], _TAPMAP[pw][dw]]
                 for dh in (0, 1) for dw in (0, 1)], axis=0))
    b = jnp.stack(bs, axis=0)
    if cpad is not None and cpad != b.shape[-1]:
        b = jnp.pad(b, ((0, 0), (0, 0), (0, cpad - b.shape[-1])))
    return b.astype(jnp.bfloat16)


def kernel(x_nchw, w1, w2, w3, w4, w5, g1, b1, g2, b2, g3, b3, g4, b4):
    x0 = x_nchw.reshape(_N, 256)
    w1p = (jnp.transpose(w1, (2, 3, 0, 1)).astype(jnp.bfloat16)
           .reshape(16, 256, 512))
    w2p = _prep_s2_weights(w2)
    w3p = _prep_s2_weights(w3)
    w4p = _prep_s2_weights(w4)
    w5p = _prep_s2_weights(w5, cpad=8)
    gb = jnp.stack([jnp.pad(a.astype(jnp.float32), (0, 512 - a.shape[0]))
                    for a in (g1, b1, g2, b2, g3, b3, g4, b4)], axis=0)

    y = pl.pallas_call(
        _gen_kernel,
        out_shape=jax.ShapeDtypeStruct((4, 8, 32 * 32 * _N), jnp.float32),
        compiler_params=pltpu.CompilerParams(
            vmem_limit_bytes=56 * 1024 * 1024),
    )(x0, w1p, w2p, w3p, w4p, w5p, gb)

    img = y[:, 0, :].reshape(2, 2, 32, 32, _N)
    img = jnp.transpose(img, (4, 2, 0, 3, 1)).reshape(_N, 64, 64)
    return img[:, None, :, :]
